# Initial kernel scaffold; baseline (speedup 1.0000x reference)
import jax
import jax.numpy as jnp
from jax.experimental import pallas as pl


def kernel(X, adj, mask, assign_matrix, H_coarse, W_inter):
    raise NotImplementedError("write your pallas kernel here")


# speedup vs baseline: 1.3686x; 1.3686x over previous
import functools

import jax
import jax.numpy as jnp
from jax import lax
from jax.experimental import pallas as pl
from jax.experimental.pallas import tpu as pltpu
from jax.experimental.pallas import tpu_sc as plsc

B, N, D, C = 16, 1000, 512, 200
K = 250
K_PAD = 256
PERCENT = 0.25
NW = 32
RPW = B * K_PAD // NW


def _phase_a(adj_ref, mask_ref, assign_ref, ai_ref, idx_ref, rm_ref):
    b = pl.program_id(0)
    adj2 = adj_ref[0]
    m = mask_ref[0][0]
    acc = adj2[:, 0:128] + adj2[:, 128:256]
    for c in range(2, 7):
        acc = acc + adj2[:, c * 128:(c + 1) * 128]
    acc = acc + jnp.concatenate(
        [adj2[:, 896:1000], jnp.zeros((N, 24), jnp.float32)], axis=1)
    s = jnp.sum(acc.T, axis=0)
    s = jnp.where(m > 0, s, -jnp.inf)

    srow = s[None, :]
    scol = s[:, None]
    ii = lax.broadcasted_iota(jnp.int32, (N, N), 1)
    jj = lax.broadcasted_iota(jnp.int32, (N, N), 0)
    loses = ((scol > srow) | ((scol == srow) & (jj < ii))).astype(jnp.bfloat16)
    ones_row = jnp.ones((1, N), jnp.bfloat16)
    rank = jnp.dot(ones_row, loses,
                   preferred_element_type=jnp.float32).astype(jnp.int32)

    rvals = lax.broadcasted_iota(jnp.int32, (K_PAD, N), 0)
    P = (rvals == rank).astype(jnp.bfloat16)

    a_inter = jnp.dot(P, assign_ref[0].astype(jnp.bfloat16),
                      preferred_element_type=jnp.float32)
    ai_ref[0] = a_inter.astype(jnp.bfloat16)

    idig = lax.broadcasted_iota(jnp.int32, (N, 2), 0)
    dsel = lax.broadcasted_iota(jnp.int32, (N, 2), 1)
    digits = jnp.where(dsel == 0, idig // 128, idig % 128).astype(jnp.bfloat16)
    pair = jnp.dot(P, digits, preferred_element_type=jnp.float32)
    idx = pair[:, 0:1] * 128.0 + pair[:, 1:2]
    idx_ref[0] = idx.astype(jnp.int32) + b * N

    k_per = jnp.ceil(PERCENT * jnp.sum(m)).astype(jnp.int32)
    rowmask = (lax.broadcasted_iota(jnp.int32, (K_PAD, 1), 0) < k_per)
    rm_ref[0] = rowmask.astype(jnp.float32)


def _sc_gather(x_hbm, idx_hbm, out_hbm, idx_v, rows_v, sem):
    wid = lax.axis_index("s") * 2 + lax.axis_index("c")
    base = wid * RPW
    pltpu.sync_copy(idx_hbm.at[pl.ds(base, RPW)], idx_v)
    pltpu.async_copy(x_hbm.at[idx_v], rows_v, sem).wait()
    pltpu.sync_copy(rows_v, out_hbm.at[pl.ds(base, RPW)])


def _phase_c(ai_ref, hf_ref, hc_ref, w_ref, rm_ref, out_ref):
    o = jnp.dot(ai_ref[0], hc_ref[0].astype(jnp.bfloat16),
                preferred_element_type=jnp.float32) + hf_ref[0]
    o = jnp.dot(o, w_ref[...])
    o = jnp.maximum(o, 0.0)
    out_ref[0] = o * rm_ref[0]


@jax.jit
def kernel(X, adj, mask, assign_matrix, H_coarse, W_inter):
    mask3 = mask.reshape(B, 1, N)

    ai, idx, rm = pl.pallas_call(
        _phase_a,
        grid=(B,),
        in_specs=[
            pl.BlockSpec((1, N, N), lambda b: (b, 0, 0)),
            pl.BlockSpec((1, 1, N), lambda b: (b, 0, 0)),
            pl.BlockSpec((1, N, C), lambda b: (b, 0, 0)),
        ],
        out_specs=[
            pl.BlockSpec((1, K_PAD, C), lambda b: (b, 0, 0)),
            pl.BlockSpec((1, K_PAD, 1), lambda b: (b, 0, 0)),
            pl.BlockSpec((1, K_PAD, 1), lambda b: (b, 0, 0)),
        ],
        out_shape=[
            jax.ShapeDtypeStruct((B, K_PAD, C), jnp.bfloat16),
            jax.ShapeDtypeStruct((B, K_PAD, 1), jnp.int32),
            jax.ShapeDtypeStruct((B, K_PAD, 1), jnp.float32),
        ],
    )(adj, mask3, assign_matrix)

    x2 = X.reshape(B * N, D)
    idx_flat = idx.reshape(B * K_PAD)

    mesh = plsc.VectorSubcoreMesh(core_axis_name="c", subcore_axis_name="s")
    gather = functools.partial(
        pl.kernel,
        out_type=jax.ShapeDtypeStruct((B * K_PAD, D), jnp.float32),
        mesh=mesh,
        scratch_types=[
            pltpu.VMEM((RPW,), jnp.int32),
            pltpu.VMEM((RPW, D), jnp.float32),
            pltpu.SemaphoreType.DMA,
        ],
    )(_sc_gather)
    h_fine = gather(x2, idx_flat)
    h_fine = h_fine.reshape(B, K_PAD, D)

    out = pl.pallas_call(
        _phase_c,
        grid=(B,),
        in_specs=[
            pl.BlockSpec((1, K_PAD, C), lambda b: (b, 0, 0)),
            pl.BlockSpec((1, K_PAD, D), lambda b: (b, 0, 0)),
            pl.BlockSpec((1, C, D), lambda b: (b, 0, 0)),
            pl.BlockSpec((D, D), lambda b: (0, 0)),
            pl.BlockSpec((1, K_PAD, 1), lambda b: (b, 0, 0)),
        ],
        out_specs=pl.BlockSpec((1, K_PAD, D), lambda b: (b, 0, 0)),
        out_shape=jax.ShapeDtypeStruct((B, K_PAD, D), jnp.float32),
    )(ai, h_fine, H_coarse, W_inter, rm)
    return out[:, :K, :]


# speedup vs baseline: 1.4006x; 1.0234x over previous
import functools

import jax
import jax.numpy as jnp
from jax import lax
from jax.experimental import pallas as pl
from jax.experimental.pallas import tpu as pltpu
from jax.experimental.pallas import tpu_sc as plsc

B, N, D, C = 16, 1000, 512, 200
K = 250
K_PAD = 256
PERCENT = 0.25
NW = 32
RPW = B * K_PAD // NW
BPS = 2


def _phase_a(adj_ref, mask_ref, assign_ref, ai_ref, idx_ref, rm_ref):
    g = pl.program_id(0)
    for t in range(BPS):
        adj2 = adj_ref[t]
        m = mask_ref[t][0]
        acc = adj2[:, 0:128] + adj2[:, 128:256]
        for c in range(2, 7):
            acc = acc + adj2[:, c * 128:(c + 1) * 128]
        acc = acc + jnp.concatenate(
            [adj2[:, 896:1000], jnp.zeros((N, 24), jnp.float32)], axis=1)
        s = jnp.sum(acc.T, axis=0)
        s = jnp.where(m > 0, s, -jnp.inf)

        srow = s[None, :]
        scol = s[:, None]
        ii = lax.broadcasted_iota(jnp.int32, (N, N), 1)
        jj = lax.broadcasted_iota(jnp.int32, (N, N), 0)
        loses = ((scol > srow)
                 | ((scol == srow) & (jj < ii))).astype(jnp.bfloat16)
        ones_row = jnp.ones((1, N), jnp.bfloat16)
        rank = jnp.dot(ones_row, loses,
                       preferred_element_type=jnp.float32).astype(jnp.int32)

        rvals = lax.broadcasted_iota(jnp.int32, (K_PAD, N), 0)
        P = (rvals == rank).astype(jnp.bfloat16)

        a_inter = jnp.dot(P, assign_ref[t].astype(jnp.bfloat16),
                          preferred_element_type=jnp.float32)
        ai_ref[t] = a_inter.astype(jnp.bfloat16)

        idig = lax.broadcasted_iota(jnp.int32, (N, 2), 0)
        dsel = lax.broadcasted_iota(jnp.int32, (N, 2), 1)
        digits = jnp.where(dsel == 0, idig // 128,
                           idig % 128).astype(jnp.bfloat16)
        pair = jnp.dot(P, digits, preferred_element_type=jnp.float32)
        idx = pair[:, 0:1] * 128.0 + pair[:, 1:2]
        idx_ref[t] = idx.astype(jnp.int32) + (g * BPS + t) * N

        k_per = jnp.ceil(PERCENT * jnp.sum(m)).astype(jnp.int32)
        rowmask = (lax.broadcasted_iota(jnp.int32, (K_PAD, 1), 0) < k_per)
        rm_ref[t] = rowmask.astype(jnp.float32)


def _sc_gather(x_hbm, idx_hbm, out_hbm, idx_v, rows_v, sem):
    wid = lax.axis_index("s") * 2 + lax.axis_index("c")
    b = wid // 2
    half = wid % 2
    pltpu.sync_copy(idx_hbm.at[pl.ds(wid * RPW, RPW)], idx_v)
    pltpu.async_copy(x_hbm.at[idx_v], rows_v, sem).wait()
    pltpu.sync_copy(rows_v, out_hbm.at[b, pl.ds(half * RPW, RPW)])


def _phase_c(ai_ref, hf_ref, hc_ref, w_ref, rm_ref, out_ref):
    o = jnp.dot(ai_ref[0], hc_ref[0].astype(jnp.bfloat16),
                preferred_element_type=jnp.float32) + hf_ref[0]
    o = jnp.dot(o, w_ref[...])
    o = jnp.maximum(o, 0.0)
    o = o * rm_ref[0]
    out_ref[0] = o[:K, :]


@jax.jit
def kernel(X, adj, mask, assign_matrix, H_coarse, W_inter):
    mask3 = mask.reshape(B, 1, N)

    ai, idx, rm = pl.pallas_call(
        _phase_a,
        grid=(B // BPS,),
        in_specs=[
            pl.BlockSpec((BPS, N, N), lambda b: (b, 0, 0)),
            pl.BlockSpec((BPS, 1, N), lambda b: (b, 0, 0)),
            pl.BlockSpec((BPS, N, C), lambda b: (b, 0, 0)),
        ],
        out_specs=[
            pl.BlockSpec((BPS, K_PAD, C), lambda b: (b, 0, 0)),
            pl.BlockSpec((BPS, K_PAD, 1), lambda b: (b, 0, 0)),
            pl.BlockSpec((BPS, K_PAD, 1), lambda b: (b, 0, 0)),
        ],
        out_shape=[
            jax.ShapeDtypeStruct((B, K_PAD, C), jnp.bfloat16),
            jax.ShapeDtypeStruct((B, K_PAD, 1), jnp.int32),
            jax.ShapeDtypeStruct((B, K_PAD, 1), jnp.float32),
        ],
    )(adj, mask3, assign_matrix)

    x2 = X.reshape(B * N, D)
    idx_flat = idx.reshape(B * K_PAD)

    mesh = plsc.VectorSubcoreMesh(core_axis_name="c", subcore_axis_name="s")
    gather = functools.partial(
        pl.kernel,
        out_type=jax.ShapeDtypeStruct((B, K_PAD, D), jnp.float32),
        mesh=mesh,
        scratch_types=[
            pltpu.VMEM((RPW,), jnp.int32),
            pltpu.VMEM((RPW, D), jnp.float32),
            pltpu.SemaphoreType.DMA,
        ],
    )(_sc_gather)
    h_fine = gather(x2, idx_flat)

    out = pl.pallas_call(
        _phase_c,
        grid=(B,),
        in_specs=[
            pl.BlockSpec((1, K_PAD, C), lambda b: (b, 0, 0)),
            pl.BlockSpec((1, K_PAD, D), lambda b: (b, 0, 0)),
            pl.BlockSpec((1, C, D), lambda b: (b, 0, 0)),
            pl.BlockSpec((D, D), lambda b: (0, 0)),
            pl.BlockSpec((1, K_PAD, 1), lambda b: (b, 0, 0)),
        ],
        out_specs=pl.BlockSpec((1, K, D), lambda b: (b, 0, 0)),
        out_shape=jax.ShapeDtypeStruct((B, K, D), jnp.float32),
    )(ai, h_fine, H_coarse, W_inter, rm)
    return out


# speedup vs baseline: 1.4829x; 1.0588x over previous
import functools

import jax
import jax.numpy as jnp
from jax import lax
from jax.experimental import pallas as pl
from jax.experimental.pallas import tpu as pltpu
from jax.experimental.pallas import tpu_sc as plsc

B, N, D, C = 16, 1000, 512, 200
K = 250
K_PAD = 256
PERCENT = 0.25
NW = 32
RPW = B * K_PAD // NW
BPS = 2
CPS = 4


def _phase_a(adj_ref, mask_ref, assign_ref, ai_ref, idx_ref, rm_ref):
    g = pl.program_id(0)
    ii = lax.broadcasted_iota(jnp.int32, (N, N), 1)
    jj = lax.broadcasted_iota(jnp.int32, (N, N), 0)
    tri = jj < ii
    for t in range(BPS):
        adj2 = adj_ref[t]
        m = mask_ref[t][0]
        acc = adj2[:, 0:128] + adj2[:, 128:256]
        for c in range(2, 7):
            acc = acc + adj2[:, c * 128:(c + 1) * 128]
        acc = acc + jnp.concatenate(
            [adj2[:, 896:1000], jnp.zeros((N, 24), jnp.float32)], axis=1)
        s = jnp.sum(acc.T, axis=0)
        s = jnp.where(m > 0, s, -jnp.inf)

        srow = s[None, :]
        scol = s[:, None]
        loses = jnp.where((scol > srow) | ((scol == srow) & tri),
                          1.0, 0.0)
        ones_row = jnp.ones((1, N), jnp.float32)
        rank = jnp.dot(ones_row, loses,
                       preferred_element_type=jnp.float32).astype(jnp.int32)

        rvals = lax.broadcasted_iota(jnp.int32, (K_PAD, N), 0)
        P = jnp.where(rvals == rank, 1.0, 0.0)

        a_inter = jnp.dot(P, assign_ref[t],
                          preferred_element_type=jnp.float32)
        ai_ref[t] = a_inter.astype(jnp.bfloat16)

        idig = lax.broadcasted_iota(jnp.int32, (N, 2), 0)
        dsel = lax.broadcasted_iota(jnp.int32, (N, 2), 1)
        digits = jnp.where(dsel == 0, idig // 128,
                           idig % 128).astype(jnp.float32)
        pair = jnp.dot(P, digits, preferred_element_type=jnp.float32)
        idx = pair[:, 0:1] * 128.0 + pair[:, 1:2]
        idx_ref[t] = idx.astype(jnp.int32) + (g * BPS + t) * N

        k_per = jnp.ceil(PERCENT * jnp.sum(m)).astype(jnp.int32)
        rowmask = (lax.broadcasted_iota(jnp.int32, (K_PAD, 1), 0) < k_per)
        rm_ref[t] = rowmask.astype(jnp.float32)


def _sc_gather(x_hbm, idx_hbm, out_hbm, idx_v, rows_v, sem):
    wid = lax.axis_index("s") * 2 + lax.axis_index("c")
    b = wid // 2
    half = wid % 2
    pltpu.sync_copy(idx_hbm.at[pl.ds(wid * RPW, RPW)], idx_v)
    pltpu.async_copy(x_hbm.at[idx_v], rows_v, sem).wait()
    pltpu.sync_copy(rows_v, out_hbm.at[b, pl.ds(half * RPW, RPW)])


def _phase_c(ai_ref, hf_ref, hc_ref, w_ref, rm_ref, out_ref):
    for t in range(CPS):
        o = jnp.dot(ai_ref[t], hc_ref[t].astype(jnp.bfloat16),
                    preferred_element_type=jnp.float32) + hf_ref[t]
        o = jnp.dot(o, w_ref[...])
        o = jnp.maximum(o, 0.0)
        o = o * rm_ref[t]
        out_ref[t] = o[:K, :]


@jax.jit
def kernel(X, adj, mask, assign_matrix, H_coarse, W_inter):
    mask3 = mask.reshape(B, 1, N)

    ai, idx, rm = pl.pallas_call(
        _phase_a,
        grid=(B // BPS,),
        in_specs=[
            pl.BlockSpec((BPS, N, N), lambda b: (b, 0, 0)),
            pl.BlockSpec((BPS, 1, N), lambda b: (b, 0, 0)),
            pl.BlockSpec((BPS, N, C), lambda b: (b, 0, 0)),
        ],
        out_specs=[
            pl.BlockSpec((BPS, K_PAD, C), lambda b: (b, 0, 0)),
            pl.BlockSpec((BPS, K_PAD, 1), lambda b: (b, 0, 0)),
            pl.BlockSpec((BPS, K_PAD, 1), lambda b: (b, 0, 0)),
        ],
        out_shape=[
            jax.ShapeDtypeStruct((B, K_PAD, C), jnp.bfloat16),
            jax.ShapeDtypeStruct((B, K_PAD, 1), jnp.int32),
            jax.ShapeDtypeStruct((B, K_PAD, 1), jnp.float32),
        ],
    )(adj, mask3, assign_matrix)

    x2 = X.reshape(B * N, D)
    idx_flat = idx.reshape(B * K_PAD)

    mesh = plsc.VectorSubcoreMesh(core_axis_name="c", subcore_axis_name="s")
    gather = functools.partial(
        pl.kernel,
        out_type=jax.ShapeDtypeStruct((B, K_PAD, D), jnp.float32),
        mesh=mesh,
        scratch_types=[
            pltpu.VMEM((RPW,), jnp.int32),
            pltpu.VMEM((RPW, D), jnp.float32),
            pltpu.SemaphoreType.DMA,
        ],
    )(_sc_gather)
    h_fine = gather(x2, idx_flat)

    out = pl.pallas_call(
        _phase_c,
        grid=(B // CPS,),
        in_specs=[
            pl.BlockSpec((CPS, K_PAD, C), lambda b: (b, 0, 0)),
            pl.BlockSpec((CPS, K_PAD, D), lambda b: (b, 0, 0)),
            pl.BlockSpec((CPS, C, D), lambda b: (b, 0, 0)),
            pl.BlockSpec((D, D), lambda b: (0, 0)),
            pl.BlockSpec((CPS, K_PAD, 1), lambda b: (b, 0, 0)),
        ],
        out_specs=pl.BlockSpec((CPS, K, D), lambda b: (b, 0, 0)),
        out_shape=jax.ShapeDtypeStruct((B, K, D), jnp.float32),
    )(ai, h_fine, H_coarse, W_inter, rm)
    return out
